# baseline (speedup 1.0000x reference)
import jax
import jax.numpy as jnp
from jax import lax
from jax.experimental import pallas as pl
from jax.experimental.pallas import tpu as pltpu
from jax.experimental.pallas import tpu_sc as plsc

_NC = 2
_NS = 16
_NW = _NC * _NS
_L = 16
_BATCH = 16384
_DIM = 64
_BPW = _BATCH // _NW
_NG = _BPW // _L

_LOG_P = (7.0376836292e-2, -1.1514610310e-1, 1.1676998740e-1,
          -1.2420140846e-1, 1.4249322787e-1, -1.6668057665e-1,
          2.0000714765e-1, -2.4999993993e-1, 3.3333331174e-1)


def _softlog(x):
    xi = lax.bitcast_convert_type(x, jnp.int32)
    e = jnp.right_shift(xi, 23) - 127
    m = lax.bitcast_convert_type((xi & 0x007FFFFF) | 0x3F800000, jnp.float32)
    big = m > 1.4142135623730951
    m = jnp.where(big, 0.5 * m, m)
    ef = (e + jnp.where(big, 1, 0)).astype(jnp.float32)
    t = m - 1.0
    p = jnp.full(x.shape, _LOG_P[0], jnp.float32)
    for c in _LOG_P[1:]:
        p = p * t + jnp.float32(c)
    z = t * t
    y = t * z * p + ef * jnp.float32(-2.12194440e-4) - 0.5 * z
    return t + y + ef * jnp.float32(0.693359375)


def _softsqrt(x):
    xi = lax.bitcast_convert_type(x, jnp.int32)
    r = lax.bitcast_convert_type(
        jnp.int32(0x5F3759DF) - jnp.right_shift(xi, 1), jnp.float32)
    for _ in range(3):
        r = r * (1.5 - 0.5 * x * r * r)
    return x * r


def _sc_body(u_hbm, i_hbm, r_hbm, b_hbm, tbl_hbm, ubias_hbm, dum_hbm,
             out_hbm,
             u_v, i_v, r_v, bu_v, bi_v, bias_v, vuf, vif, acc_v, sem, semr,
             semr2):
    wid = lax.axis_index("s") * _NC + lax.axis_index("c")
    base = pl.multiple_of(wid * _BPW, 8)

    bias_v[...] = jnp.zeros((_L,), jnp.float32)
    cps = [pltpu.async_copy(u_hbm.at[pl.ds(base, _BPW)], u_v, sem),
           pltpu.async_copy(i_hbm.at[pl.ds(base, _BPW)], i_v, sem),
           pltpu.async_copy(r_hbm.at[pl.ds(base, _BPW)], r_v, sem),
           pltpu.async_copy(b_hbm, bias_v.at[pl.ds(0, 1)], sem)]
    for cp in cps:
        cp.wait()

    def fire_group(g, semg):
        u16 = u_v[pl.ds(g * _L, _L)]
        i16 = i_v[pl.ds(g * _L, _L)]
        tu = jnp.right_shift(u16, 3)
        su = u16 & 7
        ti = jnp.right_shift(i16, 3)
        si = i16 & 7
        for j in range(_L):
            prow = g * (_L // 2) + j // 2
            psl = pl.ds((j % 2) * _DIM, _DIM)
            pltpu.async_copy(
                tbl_hbm.at[tu[j], su[j]], vuf.at[prow, psl], semg)
            pltpu.async_copy(
                tbl_hbm.at[ti[j], si[j]], vif.at[prow, psl], semg)
        return 0

    lax.fori_loop(0, _NG // 2, lambda g, c: fire_group(g, semr), 0)
    lax.fori_loop(_NG // 2, _NG, lambda g, c: fire_group(g, semr2), 0)

    cps = []
    for j in range(_BPW // 128):
        sl = pl.ds(j * 128, 128)
        cps.append(pltpu.async_copy(ubias_hbm.at[u_v.at[sl]], bu_v.at[sl], sem))
        cps.append(pltpu.async_copy(ubias_hbm.at[i_v.at[sl]], bi_v.at[sl], sem))
    hrows = _BPW // 4
    pltpu.make_async_copy(dum_hbm.at[pl.ds(0, hrows)],
                          vuf.at[pl.ds(0, hrows)], semr).wait()
    pltpu.make_async_copy(dum_hbm.at[pl.ds(0, hrows)],
                          vif.at[pl.ds(0, hrows)], semr).wait()
    for cp in cps:
        cp.wait()

    bias_s = jnp.sum(bias_v[...])
    lane = lax.iota(jnp.int32, _L)

    def compute_group(g, acc):
        def sub4(c, carry):
            squ, sqv, duv = carry
            for j in range(8):
                prow = g * (_L // 2) + c * 4 + j // 2
                pb = (j % 2) * _DIM
                au = [vuf[prow, pl.ds(pb + k * _L, _L)]
                      for k in range(_DIM // _L)]
                av = [vif[prow, pl.ds(pb + k * _L, _L)]
                      for k in range(_DIM // _L)]
                su = au[0] * au[0]
                sv = av[0] * av[0]
                dd = au[0] * av[0]
                for k in range(1, _DIM // _L):
                    su = su + au[k] * au[k]
                    sv = sv + av[k] * av[k]
                    dd = dd + au[k] * av[k]
                m = lane == (c * 8 + j)
                squ = jnp.where(m, jnp.sum(su), squ)
                sqv = jnp.where(m, jnp.sum(sv), sqv)
                duv = jnp.where(m, jnp.sum(dd), duv)
            return squ, sqv, duv

        z = jnp.zeros((_L,), jnp.float32)
        squ, sqv, duv = lax.fori_loop(0, 2, sub4, (z, z, z))

        sqd = squ + sqv - 2.0 * duv
        gs = pl.ds(g * _L, _L)
        rvec = r_v[gs]
        bu = bu_v[gs]
        bi = bi_v[gs]
        delta = 2.0 * sqd / ((1.0 - squ) * (1.0 - sqv))
        delta = jnp.maximum(delta, 1e-5)
        arg = jnp.minimum(delta * (delta + 2.0), 1e30)
        dist = _softlog(1.0 + delta + _softsqrt(arg))
        logodds = bias_s + bi + bu + dist
        p = jnp.clip(logodds, 1e-7, 1.0 - 1e-7)
        term = rvec * _softlog(p) + (1.0 - rvec) * _softlog(1.0 - p)
        return acc + term

    acc = lax.fori_loop(0, _NG // 2, compute_group,
                        jnp.zeros((_L,), jnp.float32))
    pltpu.make_async_copy(dum_hbm.at[pl.ds(0, hrows)],
                          vuf.at[pl.ds(hrows, hrows)], semr2).wait()
    pltpu.make_async_copy(dum_hbm.at[pl.ds(0, hrows)],
                          vif.at[pl.ds(hrows, hrows)], semr2).wait()
    acc = lax.fori_loop(_NG // 2, _NG, compute_group, acc)
    acc_v[...] = acc
    pltpu.sync_copy(acc_v, out_hbm.at[wid])


def _tc_reduce(p_ref, o_ref):
    o_ref[...] = jnp.full((1, 1), -jnp.sum(p_ref[...]) * (1.0 / _BATCH))


def kernel(u, i, r, user_vec, user_bias, bias):
    assert u.shape == (_BATCH,) and user_vec.shape[1] == _DIM
    tbl3 = jnp.reshape(user_vec, (user_vec.shape[0] // 8, 8, _DIM))
    dummy = jnp.zeros((_BPW // 2, 2 * _DIM), jnp.float32)

    mesh = plsc.VectorSubcoreMesh(core_axis_name="c", subcore_axis_name="s")
    sc_call = pl.kernel(
        _sc_body,
        mesh=mesh,
        compiler_params=pltpu.CompilerParams(needs_layout_passes=False),
        out_type=jax.ShapeDtypeStruct((_NW, _L), jnp.float32),
        scratch_types=[
            pltpu.VMEM((_BPW,), jnp.int32),
            pltpu.VMEM((_BPW,), jnp.int32),
            pltpu.VMEM((_BPW,), jnp.float32),
            pltpu.VMEM((_BPW,), jnp.float32),
            pltpu.VMEM((_BPW,), jnp.float32),
            pltpu.VMEM((_L,), jnp.float32),
            pltpu.VMEM((_BPW // 2, 2 * _DIM), jnp.float32),
            pltpu.VMEM((_BPW // 2, 2 * _DIM), jnp.float32),
            pltpu.VMEM((_L,), jnp.float32),
            pltpu.SemaphoreType.DMA,
            pltpu.SemaphoreType.DMA,
            pltpu.SemaphoreType.DMA,
        ],
    )
    partials = sc_call(u, i, r, bias.astype(jnp.float32), tbl3, user_bias, dummy)

    loss = pl.pallas_call(
        _tc_reduce,
        out_shape=jax.ShapeDtypeStruct((1, 1), jnp.float32),
    )(partials)
    return loss[0, 0]


# speedup vs baseline: 1.0513x; 1.0513x over previous
import jax
import jax.numpy as jnp
from jax import lax
from jax.experimental import pallas as pl
from jax.experimental.pallas import tpu as pltpu
from jax.experimental.pallas import tpu_sc as plsc

_NC = 2
_NS = 16
_NW = _NC * _NS
_L = 16
_BATCH = 16384
_DIM = 64
_BPW = _BATCH // _NW
_NG = _BPW // _L

_LOG_P = (7.0376836292e-2, -1.1514610310e-1, 1.1676998740e-1,
          -1.2420140846e-1, 1.4249322787e-1, -1.6668057665e-1,
          2.0000714765e-1, -2.4999993993e-1, 3.3333331174e-1)


def _softlog(x):
    xi = lax.bitcast_convert_type(x, jnp.int32)
    e = jnp.right_shift(xi, 23) - 127
    m = lax.bitcast_convert_type((xi & 0x007FFFFF) | 0x3F800000, jnp.float32)
    big = m > 1.4142135623730951
    m = jnp.where(big, 0.5 * m, m)
    ef = (e + jnp.where(big, 1, 0)).astype(jnp.float32)
    t = m - 1.0
    p = jnp.full(x.shape, _LOG_P[0], jnp.float32)
    for c in _LOG_P[1:]:
        p = p * t + jnp.float32(c)
    z = t * t
    y = t * z * p + ef * jnp.float32(-2.12194440e-4) - 0.5 * z
    return t + y + ef * jnp.float32(0.693359375)


def _softsqrt(x):
    xi = lax.bitcast_convert_type(x, jnp.int32)
    r = lax.bitcast_convert_type(
        jnp.int32(0x5F3759DF) - jnp.right_shift(xi, 1), jnp.float32)
    for _ in range(3):
        r = r * (1.5 - 0.5 * x * r * r)
    return x * r


def _sc_body(u_hbm, i_hbm, r_hbm, b_hbm, tbl_hbm, ubias_hbm, dum_hbm,
             out_hbm,
             u_v, i_v, r_v, bu_v, bi_v, bias_v, vuf, vif, acc_v, sem, semr):
    wid = lax.axis_index("s") * _NC + lax.axis_index("c")
    base = pl.multiple_of(wid * _BPW, 8)

    bias_v[...] = jnp.zeros((_L,), jnp.float32)
    cps = [pltpu.async_copy(u_hbm.at[pl.ds(base, _BPW)], u_v, sem),
           pltpu.async_copy(i_hbm.at[pl.ds(base, _BPW)], i_v, sem),
           pltpu.async_copy(r_hbm.at[pl.ds(base, _BPW)], r_v, sem),
           pltpu.async_copy(b_hbm, bias_v.at[pl.ds(0, 1)], sem)]
    for cp in cps:
        cp.wait()

    def fire_group(g, semg):
        u16 = u_v[pl.ds(g * _L, _L)]
        i16 = i_v[pl.ds(g * _L, _L)]
        for j in range(_L):
            prow = g * (_L // 2) + j // 2
            psl = pl.ds((j % 2) * _DIM, _DIM)
            uj = u16[j]
            ij = i16[j]
            pltpu.async_copy(
                tbl_hbm.at[jnp.right_shift(uj, 3), uj & 7],
                vuf.at[prow, psl], semg)
            pltpu.async_copy(
                tbl_hbm.at[jnp.right_shift(ij, 3), ij & 7],
                vif.at[prow, psl], semg)
        return 0

    lax.fori_loop(0, _NG, lambda g, c: fire_group(g, semr), 0)

    cps = []
    for j in range(_BPW // 128):
        sl = pl.ds(j * 128, 128)
        cps.append(pltpu.async_copy(ubias_hbm.at[u_v.at[sl]], bu_v.at[sl], sem))
        cps.append(pltpu.async_copy(ubias_hbm.at[i_v.at[sl]], bi_v.at[sl], sem))
    pltpu.make_async_copy(dum_hbm, vuf, semr).wait()
    pltpu.make_async_copy(dum_hbm, vif, semr).wait()
    for cp in cps:
        cp.wait()

    bias_s = jnp.sum(bias_v[...])
    lane = lax.iota(jnp.int32, _L)

    def compute_group(g, acc):
        def sub4(c, carry):
            squ, sqv, duv = carry
            for j in range(8):
                prow = g * (_L // 2) + c * 4 + j // 2
                pb = (j % 2) * _DIM
                au = [vuf[prow, pl.ds(pb + k * _L, _L)]
                      for k in range(_DIM // _L)]
                av = [vif[prow, pl.ds(pb + k * _L, _L)]
                      for k in range(_DIM // _L)]
                su = au[0] * au[0]
                sv = av[0] * av[0]
                dd = au[0] * av[0]
                for k in range(1, _DIM // _L):
                    su = su + au[k] * au[k]
                    sv = sv + av[k] * av[k]
                    dd = dd + au[k] * av[k]
                m = lane == (c * 8 + j)
                squ = jnp.where(m, jnp.sum(su), squ)
                sqv = jnp.where(m, jnp.sum(sv), sqv)
                duv = jnp.where(m, jnp.sum(dd), duv)
            return squ, sqv, duv

        z = jnp.zeros((_L,), jnp.float32)
        squ, sqv, duv = lax.fori_loop(0, 2, sub4, (z, z, z))

        sqd = squ + sqv - 2.0 * duv
        gs = pl.ds(g * _L, _L)
        rvec = r_v[gs]
        bu = bu_v[gs]
        bi = bi_v[gs]
        delta = 2.0 * sqd / ((1.0 - squ) * (1.0 - sqv))
        delta = jnp.maximum(delta, 1e-5)
        arg = jnp.minimum(delta * (delta + 2.0), 1e30)
        dist = _softlog(1.0 + delta + _softsqrt(arg))
        logodds = bias_s + bi + bu + dist
        p = jnp.clip(logodds, 1e-7, 1.0 - 1e-7)
        term = rvec * _softlog(p) + (1.0 - rvec) * _softlog(1.0 - p)
        return acc + term

    acc = lax.fori_loop(0, _NG, compute_group,
                        jnp.zeros((_L,), jnp.float32))
    acc_v[...] = acc
    pltpu.sync_copy(acc_v, out_hbm.at[wid])


def _tc_reduce(p_ref, o_ref):
    o_ref[...] = jnp.full((1, 1), -jnp.sum(p_ref[...]) * (1.0 / _BATCH))


def kernel(u, i, r, user_vec, user_bias, bias):
    assert u.shape == (_BATCH,) and user_vec.shape[1] == _DIM
    tbl3 = jnp.reshape(user_vec, (user_vec.shape[0] // 8, 8, _DIM))
    dummy = jnp.zeros((_BPW // 2, 2 * _DIM), jnp.float32)

    mesh = plsc.VectorSubcoreMesh(core_axis_name="c", subcore_axis_name="s")
    sc_call = pl.kernel(
        _sc_body,
        mesh=mesh,
        compiler_params=pltpu.CompilerParams(needs_layout_passes=False),
        out_type=jax.ShapeDtypeStruct((_NW, _L), jnp.float32),
        scratch_types=[
            pltpu.VMEM((_BPW,), jnp.int32),
            pltpu.VMEM((_BPW,), jnp.int32),
            pltpu.VMEM((_BPW,), jnp.float32),
            pltpu.VMEM((_BPW,), jnp.float32),
            pltpu.VMEM((_BPW,), jnp.float32),
            pltpu.VMEM((_L,), jnp.float32),
            pltpu.VMEM((_BPW // 2, 2 * _DIM), jnp.float32),
            pltpu.VMEM((_BPW // 2, 2 * _DIM), jnp.float32),
            pltpu.VMEM((_L,), jnp.float32),
            pltpu.SemaphoreType.DMA,
            pltpu.SemaphoreType.DMA,
        ],
    )
    partials = sc_call(u, i, r, bias.astype(jnp.float32), tbl3, user_bias, dummy)

    loss = pl.pallas_call(
        _tc_reduce,
        out_shape=jax.ShapeDtypeStruct((1, 1), jnp.float32),
    )(partials)
    return loss[0, 0]


# speedup vs baseline: 1.0671x; 1.0150x over previous
import jax
import jax.numpy as jnp
from jax import lax
from jax.experimental import pallas as pl
from jax.experimental.pallas import tpu as pltpu
from jax.experimental.pallas import tpu_sc as plsc

_NC = 2
_NS = 16
_NW = _NC * _NS
_L = 16
_BATCH = 16384
_DIM = 64
_BPW = _BATCH // _NW
_NG = _BPW // _L

_LOG_P = (7.0376836292e-2, -1.1514610310e-1, 1.1676998740e-1,
          -1.2420140846e-1, 1.4249322787e-1, -1.6668057665e-1,
          2.0000714765e-1, -2.4999993993e-1, 3.3333331174e-1)


def _softlog(x):
    xi = lax.bitcast_convert_type(x, jnp.int32)
    e = jnp.right_shift(xi, 23) - 127
    m = lax.bitcast_convert_type((xi & 0x007FFFFF) | 0x3F800000, jnp.float32)
    big = m > 1.4142135623730951
    m = jnp.where(big, 0.5 * m, m)
    ef = (e + jnp.where(big, 1, 0)).astype(jnp.float32)
    t = m - 1.0
    p = jnp.full(x.shape, _LOG_P[0], jnp.float32)
    for c in _LOG_P[1:]:
        p = p * t + jnp.float32(c)
    z = t * t
    y = t * z * p + ef * jnp.float32(-2.12194440e-4) - 0.5 * z
    return t + y + ef * jnp.float32(0.693359375)


def _softsqrt(x):
    xi = lax.bitcast_convert_type(x, jnp.int32)
    r = lax.bitcast_convert_type(
        jnp.int32(0x5F3759DF) - jnp.right_shift(xi, 1), jnp.float32)
    for _ in range(3):
        r = r * (1.5 - 0.5 * x * r * r)
    return x * r


def _sc_body(u_hbm, i_hbm, r_hbm, b_hbm, tbl_hbm, ubias_hbm, dum_hbm,
             out_hbm,
             u_v, i_v, r_v, bu_v, bi_v, bias_v, vuf, vif, acc_v, sem, semr):
    wid = lax.axis_index("s") * _NC + lax.axis_index("c")
    base = pl.multiple_of(wid * _BPW, 8)

    bias_v[...] = jnp.zeros((_L,), jnp.float32)
    cps = [pltpu.async_copy(u_hbm.at[pl.ds(base, _BPW)], u_v, sem),
           pltpu.async_copy(i_hbm.at[pl.ds(base, _BPW)], i_v, sem),
           pltpu.async_copy(r_hbm.at[pl.ds(base, _BPW)], r_v, sem),
           pltpu.async_copy(b_hbm, bias_v.at[pl.ds(0, 1)], sem)]
    for cp in cps:
        cp.wait()

    def fire_group(g, semg):
        u16 = u_v[pl.ds(g * _L, _L)]
        i16 = i_v[pl.ds(g * _L, _L)]
        for j in range(_L):
            prow = g * (_L // 2) + j // 2
            psl = pl.ds((j % 2) * _DIM, _DIM)
            uj = u16[j]
            ij = i16[j]
            pltpu.async_copy(
                tbl_hbm.at[jnp.right_shift(uj, 3), uj & 7],
                vuf.at[prow, psl], semg)
            pltpu.async_copy(
                tbl_hbm.at[jnp.right_shift(ij, 3), ij & 7],
                vif.at[prow, psl], semg)
        return 0

    cps = []
    for j in range(_BPW // 128):
        sl = pl.ds(j * 128, 128)
        cps.append(pltpu.async_copy(ubias_hbm.at[u_v.at[sl]], bu_v.at[sl], sem))
        cps.append(pltpu.async_copy(ubias_hbm.at[i_v.at[sl]], bi_v.at[sl], sem))

    lax.fori_loop(0, _NG, lambda g, c: fire_group(g, semr), 0)
    pltpu.make_async_copy(dum_hbm, vuf, semr).wait()
    pltpu.make_async_copy(dum_hbm, vif, semr).wait()
    for cp in cps:
        cp.wait()

    bias_s = jnp.sum(bias_v[...])
    lane = lax.iota(jnp.int32, _L)

    def compute_group(g, acc):
        def sub4(c, carry):
            squ, sqv, duv = carry
            for j in range(8):
                prow = g * (_L // 2) + c * 4 + j // 2
                pb = (j % 2) * _DIM
                au = [vuf[prow, pl.ds(pb + k * _L, _L)]
                      for k in range(_DIM // _L)]
                av = [vif[prow, pl.ds(pb + k * _L, _L)]
                      for k in range(_DIM // _L)]
                su = au[0] * au[0]
                sv = av[0] * av[0]
                dd = au[0] * av[0]
                for k in range(1, _DIM // _L):
                    su = su + au[k] * au[k]
                    sv = sv + av[k] * av[k]
                    dd = dd + au[k] * av[k]
                m = lane == (c * 8 + j)
                squ = jnp.where(m, jnp.sum(su), squ)
                sqv = jnp.where(m, jnp.sum(sv), sqv)
                duv = jnp.where(m, jnp.sum(dd), duv)
            return squ, sqv, duv

        z = jnp.zeros((_L,), jnp.float32)
        squ, sqv, duv = lax.fori_loop(0, 2, sub4, (z, z, z))

        sqd = squ + sqv - 2.0 * duv
        gs = pl.ds(g * _L, _L)
        rvec = r_v[gs]
        bu = bu_v[gs]
        bi = bi_v[gs]
        delta = 2.0 * sqd / ((1.0 - squ) * (1.0 - sqv))
        delta = jnp.maximum(delta, 1e-5)
        arg = jnp.minimum(delta * (delta + 2.0), 1e30)
        dist = _softlog(1.0 + delta + _softsqrt(arg))
        logodds = bias_s + bi + bu + dist
        p = jnp.clip(logodds, 1e-7, 1.0 - 1e-7)
        term = rvec * _softlog(p) + (1.0 - rvec) * _softlog(1.0 - p)
        return acc + term

    acc = lax.fori_loop(0, _NG, compute_group,
                        jnp.zeros((_L,), jnp.float32))
    acc_v[...] = acc
    pltpu.sync_copy(acc_v, out_hbm.at[wid])


def _tc_reduce(p_ref, o_ref):
    o_ref[...] = jnp.full((1, 1), -jnp.sum(p_ref[...]) * (1.0 / _BATCH))


def kernel(u, i, r, user_vec, user_bias, bias):
    assert u.shape == (_BATCH,) and user_vec.shape[1] == _DIM
    tbl3 = jnp.reshape(user_vec, (user_vec.shape[0] // 8, 8, _DIM))
    dummy = jnp.zeros((_BPW // 2, 2 * _DIM), jnp.float32)

    mesh = plsc.VectorSubcoreMesh(core_axis_name="c", subcore_axis_name="s")
    sc_call = pl.kernel(
        _sc_body,
        mesh=mesh,
        compiler_params=pltpu.CompilerParams(needs_layout_passes=False),
        out_type=jax.ShapeDtypeStruct((_NW, _L), jnp.float32),
        scratch_types=[
            pltpu.VMEM((_BPW,), jnp.int32),
            pltpu.VMEM((_BPW,), jnp.int32),
            pltpu.VMEM((_BPW,), jnp.float32),
            pltpu.VMEM((_BPW,), jnp.float32),
            pltpu.VMEM((_BPW,), jnp.float32),
            pltpu.VMEM((_L,), jnp.float32),
            pltpu.VMEM((_BPW // 2, 2 * _DIM), jnp.float32),
            pltpu.VMEM((_BPW // 2, 2 * _DIM), jnp.float32),
            pltpu.VMEM((_L,), jnp.float32),
            pltpu.SemaphoreType.DMA,
            pltpu.SemaphoreType.DMA,
        ],
    )
    partials = sc_call(u, i, r, bias.astype(jnp.float32), tbl3, user_bias, dummy)

    loss = pl.pallas_call(
        _tc_reduce,
        out_shape=jax.ShapeDtypeStruct((1, 1), jnp.float32),
    )(partials)
    return loss[0, 0]
